# baseline (speedup 1.0000x reference)
import functools
import jax
import jax.numpy as jnp
from jax.experimental import pallas as pl
from jax.experimental.pallas import tpu as pltpu

N = 10000
D = 128
H = 4


def _mm_body(h_ref, w_ref, o_ref):
    o_ref[...] = jnp.dot(h_ref[...], w_ref[...],
                         preferred_element_type=jnp.float32)


def _matmul(h, W):
    BN = 2000
    return pl.pallas_call(
        _mm_body,
        grid=(N // BN,),
        in_specs=[pl.BlockSpec((BN, D), lambda i: (i, 0)),
                  pl.BlockSpec((D, W.shape[1]), lambda i: (0, 0))],
        out_specs=pl.BlockSpec((BN, W.shape[1]), lambda i: (i, 0)),
        out_shape=jax.ShapeDtypeStruct((N, W.shape[1]), jnp.float32),
    )(h, W)


def _build(src, dst):
    s = jnp.concatenate([src, dst])
    d = jnp.concatenate([dst, src])
    key2 = (d * N + s) * 2
    loop_keys = (jnp.arange(N, dtype=jnp.int32) * (N + 1)) * 2 + 1
    keys = jnp.sort(jnp.concatenate([key2, loop_keys]))
    invalid = jnp.concatenate(
        [jnp.zeros((1,), bool), keys[1:] == keys[:-1]])
    half = keys >> 1
    es = half % N
    ed = half // N
    return es, ed, invalid


def kernel(h, src, dst, W_src_0, W_dst_0, attn_0, bias_0,
           W_src_1, W_dst_1, attn_1, bias_1):
    es, ed, invalid = _build(src, dst)
    es_m = jnp.where(invalid, 0, es)
    ed_m = jnp.where(invalid, N, ed)
    for W_s, W_d, a, b in ((W_src_0, W_dst_0, attn_0, bias_0),
                           (W_src_1, W_dst_1, attn_1, bias_1)):
        f = _matmul(h, jnp.concatenate([W_s, W_d], axis=1))
        fs = f[:, :H * D].reshape(N, H, D)
        fd = f[:, H * D:].reshape(N, H, D)
        e = jax.nn.leaky_relu(fs[es_m] + fd[ed_m], negative_slope=0.2)
        logits = jnp.sum(e * a[None, :, :], axis=-1)
        m = jax.ops.segment_max(logits, ed_m, num_segments=N + 1)
        ex = jnp.exp(logits - m[ed_m])
        z = jax.ops.segment_sum(ex, ed_m, num_segments=N + 1)
        alpha = ex / z[ed_m]
        msg = alpha[:, :, None] * fs[es_m]
        rst = jax.ops.segment_sum(msg, ed_m, num_segments=N + 1)[:N]
        rst = rst + b.reshape(1, H, D)
        h = jnp.mean(rst, axis=1)
    return h


# speedup vs baseline: 14.1029x; 14.1029x over previous
import functools
import numpy as np
import jax
import jax.numpy as jnp
from jax import lax
from jax.experimental import pallas as pl
from jax.experimental.pallas import tpu as pltpu
from jax.experimental.pallas import tpu_sc as plsc

N = 10000
D = 128
H = 4
HD = H * D
NW = 32
VPW = 320
NPAD = NW * VPW
EB = 128
NEG = -1e30

_GDN = lax.GatherDimensionNumbers(
    offset_dims=(), collapsed_slice_dims=(0,), start_index_map=(0,))


def _take16(v, idx):
    return lax.gather(v, idx[:, None], _GDN, (1,),
                      mode=lax.GatherScatterMode.PROMISE_IN_BOUNDS)


def _lane():
    return lax.iota(jnp.int32, 16)


def _splat(v, lane):
    return _take16(v, jnp.full((16,), lane, jnp.int32))


def _lanesum(v):
    lane = _lane()
    for sh in (1, 2, 4, 8):
        v = v + _take16(v, lane ^ sh)
    return v


def _sget(ref, i):
    return ref[pl.ds(i, 16)][0]


def _mm2_body(h_ref, ws_ref, wd_ref, fs_ref, fd_ref):
    hh = h_ref[...]
    fs_ref[...] = jnp.dot(hh, ws_ref[...], preferred_element_type=jnp.float32)
    fd_ref[...] = jnp.dot(hh, wd_ref[...], preferred_element_type=jnp.float32)


def _mm2(h, W_s, W_d):
    BN = 2000
    return pl.pallas_call(
        _mm2_body,
        grid=(N // BN,),
        in_specs=[pl.BlockSpec((BN, D), lambda i: (i, 0)),
                  pl.BlockSpec((D, HD), lambda i: (0, 0)),
                  pl.BlockSpec((D, HD), lambda i: (0, 0))],
        out_specs=[pl.BlockSpec((BN, HD), lambda i: (i, 0)),
                   pl.BlockSpec((BN, HD), lambda i: (i, 0))],
        out_shape=[jax.ShapeDtypeStruct((N, HD), jnp.float32),
                   jax.ShapeDtypeStruct((N, HD), jnp.float32)],
    )(h, W_s, W_d)


_MESH = plsc.VectorSubcoreMesh(core_axis_name="c", subcore_axis_name="s")


@functools.partial(
    pl.kernel, mesh=_MESH,
    out_type=jax.ShapeDtypeStruct((NPAD, D), jnp.float32),
    scratch_types=[
        pltpu.VMEM((EB,), jnp.int32),
        pltpu.VMEM((EB + 16,), jnp.float32),
        pltpu.VMEM((EB, HD), jnp.float32),
        pltpu.VMEM((VPW + 24,), jnp.int32),
        pltpu.VMEM((HD,), jnp.float32),
        pltpu.VMEM((HD,), jnp.float32),
        pltpu.VMEM((D,), jnp.float32),
        pltpu.VMEM((HD,), jnp.float32),
        pltpu.VMEM((16,), jnp.float32),
        pltpu.VMEM((16,), jnp.float32),
        pltpu.VMEM((D,), jnp.float32),
        pltpu.SemaphoreType.DMA,
    ])
def _edge_kernel(fs_hbm, fd_hbm, es_hbm, mk_hbm, rp_hbm, a_hbm, bm_hbm,
                 out_hbm, idx_buf, mk_buf, rows, rp_buf, fd_buf, a_buf,
                 bm_buf, acc_buf, m_buf, z_buf, orow_buf, sem):
    core = lax.axis_index("c")
    sub = lax.axis_index("s")
    wid = sub * 2 + core
    v0 = wid * VPW

    pltpu.sync_copy(rp_hbm.at[pl.ds(v0, VPW + 8)], rp_buf.at[pl.ds(0, VPW + 8)])
    pltpu.sync_copy(a_hbm, a_buf)
    pltpu.sync_copy(bm_hbm, bm_buf)

    e0 = _sget(rp_buf, 0)
    e_end = _sget(rp_buf, VPW)
    ebase = (e0 // 8) * 8
    nch = (e_end - ebase + EB - 1) // EB

    m_buf[...] = jnp.full((16,), NEG, jnp.float32)
    z_buf[...] = jnp.zeros((16,), jnp.float32)
    pltpu.sync_copy(fd_hbm.at[v0], fd_buf)

    def emit_row(v):
        zr = 1.0 / z_buf[...]
        for j in range(D // 16):
            o = jnp.zeros((16,), jnp.float32)
            for h in range(H):
                o = o + acc_buf[pl.ds(h * D + j * 16, 16)] * _splat(zr, h)
            orow_buf[pl.ds(j * 16, 16)] = o * (1.0 / H) + bm_buf[pl.ds(j * 16, 16)]
        pltpu.sync_copy(orow_buf, out_hbm.at[v])

    def chunk_body(c, vstate):
        cbase = ebase + c * EB
        pltpu.sync_copy(es_hbm.at[pl.ds(cbase, EB)], idx_buf)
        pltpu.sync_copy(mk_hbm.at[pl.ds(cbase, EB)], mk_buf.at[pl.ds(0, EB)])
        pltpu.async_copy(fs_hbm.at[idx_buf], rows, sem).wait()
        lo = jnp.maximum(e0 - cbase, 0)
        hi = jnp.minimum(e_end - cbase, EB)

        def edge_body(i, vs):
            v, send = vs

            def fin(v, send):
                emit_row(v)
                vn = v + 1
                m_buf[...] = jnp.full((16,), NEG, jnp.float32)
                z_buf[...] = jnp.zeros((16,), jnp.float32)
                pltpu.sync_copy(fd_hbm.at[vn], fd_buf)
                return vn, _sget(rp_buf, vn - v0 + 1)

            def nofin(v, send):
                return v, send

            v, send = lax.cond(cbase + i == send, fin, nofin, v, send)

            ps = []
            for h in range(H):
                p = jnp.zeros((16,), jnp.float32)
                for j in range(D // 16):
                    sl = pl.ds(h * D + j * 16, 16)
                    x = rows[i, sl] + fd_buf[sl]
                    t = jnp.maximum(x, 0.2 * x)
                    p = p + t * a_buf[sl]
                ps.append(_lanesum(p))
            lane4 = _lane() % 4
            L = jnp.where(lane4 < 2,
                          jnp.where(lane4 == 0, ps[0], ps[1]),
                          jnp.where(lane4 == 2, ps[2], ps[3]))
            mk = _splat(mk_buf[pl.ds(i, 16)], 0)
            m_old = m_buf[...]
            mn = jnp.maximum(m_old, L)
            ex = jnp.exp(L - mn) * mk
            f = jnp.exp(m_old - mn)
            z_buf[...] = z_buf[...] * f + ex
            m_buf[...] = mn
            for h in range(H):
                fh = _splat(f, h)
                eh = _splat(ex, h)
                for j in range(D // 16):
                    sl = pl.ds(h * D + j * 16, 16)
                    acc_buf[sl] = acc_buf[sl] * fh + eh * rows[i, sl]
            return v, send

        return lax.fori_loop(lo, hi, edge_body, vstate)

    v_fin, _ = lax.fori_loop(0, nch, chunk_body, (v0 + 0, _sget(rp_buf, 1)))
    emit_row(v_fin)


def _build(src, dst):
    s = jnp.concatenate([src, dst])
    d = jnp.concatenate([dst, src])
    key2 = (d * N + s) * 2
    loop_keys = (jnp.arange(N, dtype=jnp.int32) * (N + 1)) * 2 + 1
    keys = jnp.sort(jnp.concatenate([key2, loop_keys]))
    invalid = jnp.concatenate(
        [jnp.zeros((1,), bool), keys[1:] == keys[:-1]])
    half = keys >> 1
    es = (half % N).astype(jnp.int32)
    ed = (half // N).astype(jnp.int32)
    maskf = jnp.where(invalid, 0.0, 1.0).astype(jnp.float32)
    rp = jnp.searchsorted(ed, jnp.arange(NPAD + 32, dtype=jnp.int32),
                          side='left').astype(jnp.int32)
    e_tot = es.shape[0]
    pad = (-e_tot) % 8 + EB
    es = jnp.concatenate([es, jnp.zeros((pad,), jnp.int32)])
    maskf = jnp.concatenate([maskf, jnp.zeros((pad,), jnp.float32)])
    return es, maskf, rp


def kernel(h, src, dst, W_src_0, W_dst_0, attn_0, bias_0,
           W_src_1, W_dst_1, attn_1, bias_1):
    es, maskf, rp = _build(src, dst)
    for W_s, W_d, a, b in ((W_src_0, W_dst_0, attn_0, bias_0),
                           (W_src_1, W_dst_1, attn_1, bias_1)):
        fs, fd = _mm2(h, W_s, W_d)
        a_flat = a.reshape(HD)
        bm = b.reshape(H, D).mean(axis=0)
        out = _edge_kernel(fs, fd, es, maskf, rp, a_flat, bm)
        h = out[:N]
    return h


# speedup vs baseline: 26.7613x; 1.8976x over previous
import functools
import numpy as np
import jax
import jax.numpy as jnp
from jax import lax
from jax.experimental import pallas as pl
from jax.experimental.pallas import tpu as pltpu
from jax.experimental.pallas import tpu_sc as plsc

N = 10000
D = 128
H = 4
HD = H * D
NW = 32
VPW = 320
NPAD = NW * VPW
EB = 128
NEG = -1e30

_GDN = lax.GatherDimensionNumbers(
    offset_dims=(), collapsed_slice_dims=(0,), start_index_map=(0,))


def _take16(v, idx):
    return lax.gather(v, idx[:, None], _GDN, (1,),
                      mode=lax.GatherScatterMode.PROMISE_IN_BOUNDS)


def _lane():
    return lax.iota(jnp.int32, 16)


def _splat(v, lane):
    return _take16(v, jnp.full((16,), lane, jnp.int32))


def _lanesum(v):
    lane = _lane()
    for sh in (1, 2, 4, 8):
        v = v + _take16(v, lane ^ sh)
    return v


def _sget(ref, i):
    return ref[pl.ds(i, 16)][0]


def _mm2_body(h_ref, ws_ref, wd_ref, fs_ref, fd_ref):
    hh = h_ref[...]
    fs_ref[...] = jnp.dot(hh, ws_ref[...], preferred_element_type=jnp.float32)
    fd_ref[...] = jnp.dot(hh, wd_ref[...], preferred_element_type=jnp.float32)


def _mm2(h, W_s, W_d):
    BN = 2000
    return pl.pallas_call(
        _mm2_body,
        grid=(N // BN,),
        in_specs=[pl.BlockSpec((BN, D), lambda i: (i, 0)),
                  pl.BlockSpec((D, HD), lambda i: (0, 0)),
                  pl.BlockSpec((D, HD), lambda i: (0, 0))],
        out_specs=[pl.BlockSpec((BN, HD), lambda i: (i, 0)),
                   pl.BlockSpec((BN, HD), lambda i: (i, 0))],
        out_shape=[jax.ShapeDtypeStruct((N, HD), jnp.float32),
                   jax.ShapeDtypeStruct((N, HD), jnp.float32)],
    )(h, W_s, W_d)


_MESH = plsc.VectorSubcoreMesh(core_axis_name="c", subcore_axis_name="s")


@functools.partial(
    pl.kernel, mesh=_MESH,
    out_type=jax.ShapeDtypeStruct((NPAD, D), jnp.float32),
    scratch_types=[
        pltpu.VMEM((EB,), jnp.int32),
        pltpu.VMEM((EB + 16,), jnp.float32),
        pltpu.VMEM((EB, HD), jnp.float32),
        pltpu.VMEM((VPW + 24,), jnp.int32),
        pltpu.VMEM((HD,), jnp.float32),
        pltpu.VMEM((HD,), jnp.float32),
        pltpu.VMEM((D,), jnp.float32),
        pltpu.VMEM((HD,), jnp.float32),
        pltpu.VMEM((16,), jnp.float32),
        pltpu.VMEM((16,), jnp.float32),
        pltpu.VMEM((D,), jnp.float32),
        pltpu.VMEM((16 * EB,), jnp.float32),
        pltpu.SemaphoreType.DMA,
    ])
def _edge_kernel(fs_hbm, fd_hbm, es_hbm, mk_hbm, rp_hbm, a_hbm, bm_hbm,
                 out_hbm, idx_buf, mk_buf, rows, rp_buf, fd_buf, a_buf,
                 bm_buf, acc_buf, m_buf, z_buf, orow_buf, larr, sem):
    core = lax.axis_index("c")
    sub = lax.axis_index("s")
    wid = sub * 2 + core
    v0 = wid * VPW

    pltpu.sync_copy(rp_hbm.at[pl.ds(v0, VPW + 8)], rp_buf.at[pl.ds(0, VPW + 8)])
    pltpu.sync_copy(a_hbm, a_buf)
    pltpu.sync_copy(bm_hbm, bm_buf)

    e0 = _sget(rp_buf, 0)
    e_end = _sget(rp_buf, VPW)
    ebase = (e0 // 8) * 8
    nch = (e_end - ebase + EB - 1) // EB

    m_buf[...] = jnp.full((16,), NEG, jnp.float32)
    z_buf[...] = jnp.zeros((16,), jnp.float32)
    pltpu.sync_copy(fd_hbm.at[v0], fd_buf)

    def emit_row(v):
        zr = 1.0 / z_buf[...]
        for j in range(D // 16):
            o = jnp.zeros((16,), jnp.float32)
            for h in range(H):
                o = o + acc_buf[pl.ds(h * D + j * 16, 16)] * _splat(zr, h)
            orow_buf[pl.ds(j * 16, 16)] = o * (1.0 / H) + bm_buf[pl.ds(j * 16, 16)]
        pltpu.sync_copy(orow_buf, out_hbm.at[v])

    def sub_body(_, st):
        p, v, send, c = st

        def work(st):
            p, v, send, c = st

            def loadch(c):
                cbase = ebase + c * EB
                pltpu.sync_copy(es_hbm.at[pl.ds(cbase, EB)], idx_buf)
                pltpu.sync_copy(mk_hbm.at[pl.ds(cbase, EB)],
                                mk_buf.at[pl.ds(0, EB)])
                pltpu.async_copy(fs_hbm.at[idx_buf], rows, sem).wait()
                return c + 1

            c = lax.cond(p >= ebase + c * EB, loadch, lambda c: c, c)
            cbase = ebase + (c - 1) * EB
            sub_hi = jnp.minimum(jnp.minimum(send, cbase + EB), e_end)
            el_lo = p - cbase
            el_hi = sub_hi - cbase

            def Lbody(el, _):
                ps = []
                for h in range(H):
                    pv = jnp.zeros((16,), jnp.float32)
                    for j in range(D // 16):
                        sl = pl.ds(h * D + j * 16, 16)
                        x = rows[el, sl] + fd_buf[sl]
                        t = jnp.maximum(x, 0.2 * x)
                        pv = pv + t * a_buf[sl]
                    ps.append(_lanesum(pv))
                lane4 = _lane() % 4
                L = jnp.where(lane4 < 2,
                              jnp.where(lane4 == 0, ps[0], ps[1]),
                              jnp.where(lane4 == 2, ps[2], ps[3]))
                larr[pl.ds(el * 16, 16)] = L
                return 0
            lax.fori_loop(el_lo, el_hi, Lbody, 0)

            m_old = m_buf[...]

            def Smax(el, mm):
                return jnp.maximum(mm, larr[pl.ds(el * 16, 16)])
            m_sub = lax.fori_loop(el_lo, el_hi, Smax,
                                  jnp.full((16,), NEG, jnp.float32))
            mn = jnp.maximum(m_old, m_sub)
            f = jnp.exp(m_old - mn)

            def Sex(el, zz):
                lv = larr[pl.ds(el * 16, 16)]
                mk = _splat(mk_buf[pl.ds(el, 16)], 0)
                ex = jnp.exp(lv - mn) * mk
                larr[pl.ds(el * 16, 16)] = ex
                return zz + ex
            z_sub = lax.fori_loop(el_lo, el_hi, Sex,
                                  jnp.zeros((16,), jnp.float32))
            z_buf[...] = z_buf[...] * f + z_sub
            m_buf[...] = mn

            accs = tuple(acc_buf[pl.ds(k * 16, 16)] * _splat(f, k // (D // 16))
                         for k in range(HD // 16))

            def Mbody(el, accs):
                exv = larr[pl.ds(el * 16, 16)]
                new = []
                for h in range(H):
                    eh = _splat(exv, h)
                    for j in range(D // 16):
                        k = h * (D // 16) + j
                        sl = pl.ds(h * D + j * 16, 16)
                        new.append(accs[k] + eh * rows[el, sl])
                return tuple(new)
            accs = lax.fori_loop(el_lo, el_hi, Mbody, accs)
            for k in range(HD // 16):
                acc_buf[pl.ds(k * 16, 16)] = accs[k]

            def fin(v, send):
                emit_row(v)
                vn = v + 1
                m_buf[...] = jnp.full((16,), NEG, jnp.float32)
                pltpu.sync_copy(fd_hbm.at[jnp.minimum(vn, N - 1)], fd_buf)
                return vn, _sget(rp_buf, vn - v0 + 1)

            def nofin(v, send):
                return v, send

            v, send = lax.cond(sub_hi == send, fin, nofin, v, send)
            return (sub_hi, v, send, c)

        return lax.cond(p < e_end, work, lambda st: st, st)

    lax.fori_loop(0, VPW + nch + 2, sub_body,
                  (e0, v0 + 0, _sget(rp_buf, 1), 0))


def _build(src, dst):
    s = jnp.concatenate([src, dst])
    d = jnp.concatenate([dst, src])
    key2 = (d * N + s) * 2
    loop_keys = (jnp.arange(N, dtype=jnp.int32) * (N + 1)) * 2 + 1
    keys = jnp.sort(jnp.concatenate([key2, loop_keys]))
    invalid = jnp.concatenate(
        [jnp.zeros((1,), bool), keys[1:] == keys[:-1]])
    half = keys >> 1
    es = (half % N).astype(jnp.int32)
    ed = (half // N).astype(jnp.int32)
    maskf = jnp.where(invalid, 0.0, 1.0).astype(jnp.float32)
    rp = jnp.searchsorted(ed, jnp.arange(NPAD + 32, dtype=jnp.int32),
                          side='left').astype(jnp.int32)
    e_tot = es.shape[0]
    pad = (-e_tot) % 8 + EB
    es = jnp.concatenate([es, jnp.zeros((pad,), jnp.int32)])
    maskf = jnp.concatenate([maskf, jnp.zeros((pad,), jnp.float32)])
    return es, maskf, rp


def kernel(h, src, dst, W_src_0, W_dst_0, attn_0, bias_0,
           W_src_1, W_dst_1, attn_1, bias_1):
    es, maskf, rp = _build(src, dst)
    for W_s, W_d, a, b in ((W_src_0, W_dst_0, attn_0, bias_0),
                           (W_src_1, W_dst_1, attn_1, bias_1)):
        fs, fd = _mm2(h, W_s, W_d)
        a_flat = a.reshape(HD)
        bm = b.reshape(H, D).mean(axis=0)
        out = _edge_kernel(fs, fd, es, maskf, rp, a_flat, bm)
        h = out[:N]
    return h


# speedup vs baseline: 31.0102x; 1.1588x over previous
import functools
import numpy as np
import jax
import jax.numpy as jnp
from jax import lax
from jax.experimental import pallas as pl
from jax.experimental.pallas import tpu as pltpu
from jax.experimental.pallas import tpu_sc as plsc

N = 10000
D = 128
H = 4
HD = H * D
NW = 32
VPW = 320
NPAD = NW * VPW
EB = 128
NEG = -1e30

_GDN = lax.GatherDimensionNumbers(
    offset_dims=(), collapsed_slice_dims=(0,), start_index_map=(0,))


def _take16(v, idx):
    return lax.gather(v, idx[:, None], _GDN, (1,),
                      mode=lax.GatherScatterMode.PROMISE_IN_BOUNDS)


def _lane():
    return lax.iota(jnp.int32, 16)


def _splat(v, lane):
    return _take16(v, jnp.full((16,), lane, jnp.int32))


def _lanesum(v):
    lane = _lane()
    for sh in (1, 2, 4, 8):
        v = v + _take16(v, lane ^ sh)
    return v


def _sget(ref, i):
    return ref[pl.ds(i, 16)][0]


def _mm2_body(h_ref, ws_ref, wd_ref, fs_ref, fd_ref):
    hh = h_ref[...]
    fs_ref[...] = jnp.dot(hh, ws_ref[...], preferred_element_type=jnp.float32)
    fd_ref[...] = jnp.dot(hh, wd_ref[...], preferred_element_type=jnp.float32)


def _mm2(h, W_s, W_d):
    BN = 2000
    return pl.pallas_call(
        _mm2_body,
        grid=(N // BN,),
        in_specs=[pl.BlockSpec((BN, D), lambda i: (i, 0)),
                  pl.BlockSpec((D, HD), lambda i: (0, 0)),
                  pl.BlockSpec((D, HD), lambda i: (0, 0))],
        out_specs=[pl.BlockSpec((BN, HD), lambda i: (i, 0)),
                   pl.BlockSpec((BN, HD), lambda i: (i, 0))],
        out_shape=[jax.ShapeDtypeStruct((N, HD), jnp.float32),
                   jax.ShapeDtypeStruct((N, HD), jnp.float32)],
    )(h, W_s, W_d)


_MESH = plsc.VectorSubcoreMesh(core_axis_name="c", subcore_axis_name="s")


@functools.partial(
    pl.kernel, mesh=_MESH,
    out_type=jax.ShapeDtypeStruct((NPAD, D), jnp.float32),
    scratch_types=[
        pltpu.VMEM((EB,), jnp.int32),
        pltpu.VMEM((EB + 16,), jnp.float32),
        pltpu.VMEM((EB + 1, HD), jnp.float32),
        pltpu.VMEM((VPW + 24,), jnp.int32),
        pltpu.VMEM((HD,), jnp.float32),
        pltpu.VMEM((HD,), jnp.float32),
        pltpu.VMEM((D,), jnp.float32),
        pltpu.VMEM((HD,), jnp.float32),
        pltpu.VMEM((16,), jnp.float32),
        pltpu.VMEM((16,), jnp.float32),
        pltpu.VMEM((D,), jnp.float32),
        pltpu.VMEM((16 * (EB + 1),), jnp.float32),
        pltpu.SemaphoreType.DMA,
    ])
def _edge_kernel(fs_hbm, fd_hbm, es_hbm, mk_hbm, rp_hbm, a_hbm, bm_hbm,
                 out_hbm, idx_buf, mk_buf, rows, rp_buf, fd_buf, a_buf,
                 bm_buf, acc_buf, m_buf, z_buf, orow_buf, larr, sem):
    core = lax.axis_index("c")
    sub = lax.axis_index("s")
    wid = sub * 2 + core
    v0 = wid * VPW

    pltpu.sync_copy(rp_hbm.at[pl.ds(v0, VPW + 8)], rp_buf.at[pl.ds(0, VPW + 8)])
    pltpu.sync_copy(a_hbm, a_buf)
    pltpu.sync_copy(bm_hbm, bm_buf)

    e0 = _sget(rp_buf, 0)
    e_end = _sget(rp_buf, VPW)
    ebase = (e0 // 8) * 8
    nch = (e_end - ebase + EB - 1) // EB

    m_buf[...] = jnp.full((16,), NEG, jnp.float32)
    z_buf[...] = jnp.zeros((16,), jnp.float32)
    pltpu.sync_copy(fd_hbm.at[v0], fd_buf)

    def emit_row(v):
        zr = 1.0 / z_buf[...]
        for j in range(D // 16):
            o = jnp.zeros((16,), jnp.float32)
            for h in range(H):
                o = o + acc_buf[pl.ds(h * D + j * 16, 16)] * _splat(zr, h)
            orow_buf[pl.ds(j * 16, 16)] = o * (1.0 / H) + bm_buf[pl.ds(j * 16, 16)]
        pltpu.sync_copy(orow_buf, out_hbm.at[v])

    def sub_body(_, st):
        p, v, send, c = st

        def work(st):
            p, v, send, c = st

            def loadch(c):
                cbase = ebase + c * EB
                pltpu.sync_copy(es_hbm.at[pl.ds(cbase, EB)], idx_buf)
                pltpu.sync_copy(mk_hbm.at[pl.ds(cbase, EB)],
                                mk_buf.at[pl.ds(0, EB)])
                pltpu.async_copy(fs_hbm.at[idx_buf], rows.at[pl.ds(0, EB)],
                                 sem).wait()
                return c + 1

            c = lax.cond(p >= ebase + c * EB, loadch, lambda c: c, c)
            cbase = ebase + (c - 1) * EB
            sub_hi = jnp.minimum(jnp.minimum(send, cbase + EB), e_end)
            el_lo = p - cbase
            el_hi = sub_hi - cbase

            def Lbody(k, _):
                el = el_lo + 2 * k
                ps0, ps1 = [], []
                for h in range(H):
                    pv0 = jnp.zeros((16,), jnp.float32)
                    pv1 = jnp.zeros((16,), jnp.float32)
                    for j in range(D // 16):
                        sl = pl.ds(h * D + j * 16, 16)
                        fdv = fd_buf[sl]
                        av = a_buf[sl]
                        x0 = rows[el, sl] + fdv
                        x1 = rows[el + 1, sl] + fdv
                        pv0 = pv0 + jnp.maximum(x0, 0.2 * x0) * av
                        pv1 = pv1 + jnp.maximum(x1, 0.2 * x1) * av
                    ps0.append(_lanesum(pv0))
                    ps1.append(_lanesum(pv1))
                lane4 = _lane() % 4
                for off, ps in ((0, ps0), (1, ps1)):
                    L = jnp.where(lane4 < 2,
                                  jnp.where(lane4 == 0, ps[0], ps[1]),
                                  jnp.where(lane4 == 2, ps[2], ps[3]))
                    larr[pl.ds((el + off) * 16, 16)] = L
                return 0
            lax.fori_loop(0, (el_hi - el_lo + 1) // 2, Lbody, 0)

            m_old = m_buf[...]

            def Smax(el, mm):
                return jnp.maximum(mm, larr[pl.ds(el * 16, 16)])
            m_sub = lax.fori_loop(el_lo, el_hi, Smax,
                                  jnp.full((16,), NEG, jnp.float32))
            mn = jnp.maximum(m_old, m_sub)
            f = jnp.exp(m_old - mn)

            def Sex(el, zz):
                lv = larr[pl.ds(el * 16, 16)]
                mk = _splat(mk_buf[pl.ds(el, 16)], 0)
                ex = jnp.exp(lv - mn) * mk
                larr[pl.ds(el * 16, 16)] = ex
                return zz + ex
            z_sub = lax.fori_loop(el_lo, el_hi, Sex,
                                  jnp.zeros((16,), jnp.float32))
            z_buf[...] = z_buf[...] * f + z_sub
            m_buf[...] = mn

            accs = tuple(acc_buf[pl.ds(k * 16, 16)] * _splat(f, k // (D // 16))
                         for k in range(HD // 16))

            def Mbody(el, accs):
                exv = larr[pl.ds(el * 16, 16)]
                new = []
                for h in range(H):
                    eh = _splat(exv, h)
                    for j in range(D // 16):
                        k = h * (D // 16) + j
                        sl = pl.ds(h * D + j * 16, 16)
                        new.append(accs[k] + eh * rows[el, sl])
                return tuple(new)
            accs = lax.fori_loop(el_lo, el_hi, Mbody, accs)
            for k in range(HD // 16):
                acc_buf[pl.ds(k * 16, 16)] = accs[k]

            def fin(v, send):
                emit_row(v)
                vn = v + 1
                m_buf[...] = jnp.full((16,), NEG, jnp.float32)
                pltpu.sync_copy(fd_hbm.at[jnp.minimum(vn, N - 1)], fd_buf)
                return vn, _sget(rp_buf, vn - v0 + 1)

            def nofin(v, send):
                return v, send

            v, send = lax.cond(sub_hi == send, fin, nofin, v, send)
            return (sub_hi, v, send, c)

        return lax.cond(p < e_end, work, lambda st: st, st)

    lax.fori_loop(0, VPW + nch + 2, sub_body,
                  (e0, v0 + 0, _sget(rp_buf, 1), 0))


def _build(src, dst):
    s = jnp.concatenate([src, dst])
    d = jnp.concatenate([dst, src])
    key2 = (d * N + s) * 2
    loop_keys = (jnp.arange(N, dtype=jnp.int32) * (N + 1)) * 2 + 1
    keys = jnp.sort(jnp.concatenate([key2, loop_keys]))
    invalid = jnp.concatenate(
        [jnp.zeros((1,), bool), keys[1:] == keys[:-1]])
    half = keys >> 1
    es = (half % N).astype(jnp.int32)
    ed = (half // N).astype(jnp.int32)
    maskf = jnp.where(invalid, 0.0, 1.0).astype(jnp.float32)
    rp = jnp.searchsorted(ed, jnp.arange(NPAD + 32, dtype=jnp.int32),
                          side='left').astype(jnp.int32)
    e_tot = es.shape[0]
    pad = (-e_tot) % 8 + EB
    es = jnp.concatenate([es, jnp.zeros((pad,), jnp.int32)])
    maskf = jnp.concatenate([maskf, jnp.zeros((pad,), jnp.float32)])
    return es, maskf, rp


def kernel(h, src, dst, W_src_0, W_dst_0, attn_0, bias_0,
           W_src_1, W_dst_1, attn_1, bias_1):
    es, maskf, rp = _build(src, dst)
    for W_s, W_d, a, b in ((W_src_0, W_dst_0, attn_0, bias_0),
                           (W_src_1, W_dst_1, attn_1, bias_1)):
        fs, fd = _mm2(h, W_s, W_d)
        a_flat = a.reshape(HD)
        bm = b.reshape(H, D).mean(axis=0)
        out = _edge_kernel(fs, fd, es, maskf, rp, a_flat, bm)
        h = out[:N]
    return h
